# baseline (speedup 1.0000x reference)
import functools

import jax
import jax.numpy as jnp
from jax import lax
from jax.experimental import pallas as pl
from jax.experimental.pallas import tpu as pltpu
from jax.experimental.pallas import tpu_sc as plsc

_D = 64
_NC = 2
_NS = 16
_NW = _NC * _NS
_CHUNK = 128
_SEL_BLK = 512


def _sc_gather(t2, idx):
    nw, n_chunks, chunk = idx.shape
    b = nw * n_chunks * chunk
    b_per_w = n_chunks * chunk

    mesh = plsc.VectorSubcoreMesh(core_axis_name="c", subcore_axis_name="s")

    @functools.partial(
        pl.kernel,
        mesh=mesh,
        out_type=jax.ShapeDtypeStruct((b, 128), t2.dtype),
        scratch_types=[
            pltpu.VMEM((n_chunks, chunk), jnp.int32),
            pltpu.VMEM((b_per_w, 128), jnp.float32),
            pltpu.SemaphoreType.DMA,
        ],
    )
    def gather_kernel(t2_hbm, idx_hbm, out_hbm, idx_v, rows_v, sem):
        wid = lax.axis_index("s") * _NC + lax.axis_index("c")
        base = wid * b_per_w
        pltpu.sync_copy(idx_hbm.at[wid], idx_v)
        copies = [
            pltpu.async_copy(
                t2_hbm.at[idx_v.at[j]],
                rows_v.at[pl.ds(j * chunk, chunk)],
                sem,
            )
            for j in range(n_chunks)
        ]
        for c in copies:
            c.wait()
        pltpu.sync_copy(rows_v, out_hbm.at[pl.ds(base, b_per_w)])

    return gather_kernel(t2, idx)


def _half_select(g, idx64):
    b = g.shape[0]

    def body(idx_ref, g_ref, out_ref):
        sel = (idx_ref[...] & 1) != 0
        out_ref[...] = jnp.where(sel, g_ref[:, _D:], g_ref[:, :_D])

    return pl.pallas_call(
        body,
        grid=(b // _SEL_BLK,),
        in_specs=[
            pl.BlockSpec((_SEL_BLK, _D), lambda i: (i, 0)),
            pl.BlockSpec((_SEL_BLK, 2 * _D), lambda i: (i, 0)),
        ],
        out_specs=pl.BlockSpec((_SEL_BLK, _D), lambda i: (i, 0)),
        out_shape=jax.ShapeDtypeStruct((b, _D), g.dtype),
    )(idx64, g)


def kernel(batch, table):
    b = batch.shape[0]
    b_per_w = b // _NW
    n_chunks = b_per_w // _CHUNK
    idx = batch.astype(jnp.int32)
    t2 = table.reshape(table.shape[0] // 2, 2 * _D)
    g = _sc_gather(t2, (idx >> 1).reshape(_NW, n_chunks, _CHUNK))
    idx64 = jnp.broadcast_to(idx[:, None], (b, _D))
    out = _half_select(g, idx64)
    return out.reshape(b, 1, _D)


# speedup vs baseline: 1.0421x; 1.0421x over previous
import dataclasses
import functools

import jax
import jax.numpy as jnp
from jax import lax
from jax.experimental import pallas as pl
from jax.experimental.pallas import tpu as pltpu
from jax.experimental.pallas import tpu_sc as plsc

_D = 64
_NC = 2
_NS = 16
_NW = _NC * _NS
_CHUNK = 128


def kernel(batch, table):
    b = batch.shape[0]
    b_per_w = b // _NW
    n_chunks = b_per_w // _CHUNK
    idx = batch.astype(jnp.int32).reshape(_NW, n_chunks, _CHUNK)

    mesh = plsc.VectorSubcoreMesh(core_axis_name="c", subcore_axis_name="s")
    cp = pltpu.CompilerParams(use_tc_tiling_on_sc=False)

    @functools.partial(
        pl.kernel,
        mesh=mesh,
        out_type=jax.ShapeDtypeStruct((b, _D), table.dtype),
        scratch_types=[
            pltpu.VMEM((n_chunks, _CHUNK), jnp.int32),
            pltpu.VMEM((b_per_w, _D), jnp.float32),
            pltpu.SemaphoreType.DMA,
        ],
        compiler_params=cp,
    )
    def gather_kernel(table_hbm, idx_hbm, out_hbm, idx_v, rows_v, sem):
        wid = lax.axis_index("s") * _NC + lax.axis_index("c")
        base = wid * b_per_w
        pltpu.sync_copy(idx_hbm.at[wid], idx_v)
        copies = [
            pltpu.async_copy(
                table_hbm.at[idx_v.at[j]],
                rows_v.at[pl.ds(j * _CHUNK, _CHUNK)],
                sem,
            )
            for j in range(n_chunks)
        ]
        for c in copies:
            c.wait()
        pltpu.sync_copy(rows_v, out_hbm.at[pl.ds(base, b_per_w)])

    out = gather_kernel(table, idx)
    return out.reshape(b, 1, _D)


# speedup vs baseline: 2.1700x; 2.0824x over previous
import functools

import jax
import jax.numpy as jnp
from jax import lax
from jax.experimental import pallas as pl
from jax.experimental.pallas import tpu as pltpu
from jax.experimental.pallas import tpu_sc as plsc

_D = 64
_NC = 2
_NS = 16
_NW = _NC * _NS
_NBUF = 4


def kernel(batch, table):
    b = batch.shape[0]
    per_w = b // _NW
    idx = batch.astype(jnp.int32).reshape(_NW, per_w)

    tt = table.T

    mesh = plsc.VectorSubcoreMesh(core_axis_name="c", subcore_axis_name="s")
    cp = pltpu.CompilerParams(
        needs_layout_passes=False, disable_bounds_checks=True
    )

    @functools.partial(
        pl.kernel,
        mesh=mesh,
        compiler_params=cp,
        out_type=jax.ShapeDtypeStruct((b, _D), table.dtype),
        scratch_types=[
            pltpu.VMEM((per_w,), jnp.int32),
            pltpu.VMEM((_NBUF, _D, 128), jnp.float32),
            pltpu.VMEM((per_w // 2, _D), jnp.float32),
            pltpu.SemaphoreType.DMA((_NBUF,)),
        ],
    )
    def gather_kernel(tt_hbm, idx_hbm, out_hbm, idx_v, tiles_v, out_v, sems):
        wid = lax.axis_index("s") * _NC + lax.axis_index("c")
        pltpu.sync_copy(idx_hbm.at[wid], idx_v)

        iota16 = lax.iota(jnp.int32, 16)
        half_w = per_w // 2

        for half in range(2):

            @pl.loop(0, half_w, step=16)
            def _(j0, half=half):
                jv = idx_v[pl.ds(j0 + half * half_w, 16)]
                for h in range(4):
                    for t in range(_NBUF):
                        r = jv[h * _NBUF + t]
                        c128 = pl.multiple_of((r >> 7) * 128, 128)
                        pltpu.async_copy(
                            tt_hbm.at[:, pl.ds(c128, 128)],
                            tiles_v.at[t],
                            sems.at[t],
                        )
                    for t in range(_NBUF):
                        pltpu.make_async_copy(
                            tt_hbm.at[:, pl.ds(0, 128)],
                            tiles_v.at[t],
                            sems.at[t],
                        ).wait()
                        lane = jnp.broadcast_to(jv[h * _NBUF + t] & 127, (16,))
                        for q in range(4):
                            vals = plsc.load_gather(
                                tiles_v.at[t], [iota16 + 16 * q, lane]
                            )
                            out_v[j0 + h * _NBUF + t, pl.ds(16 * q, 16)] = vals

            pltpu.sync_copy(
                out_v,
                out_hbm.at[pl.ds(wid * per_w + half * half_w, half_w)],
            )

    out = gather_kernel(tt, idx)
    return out.reshape(b, 1, _D)


# speedup vs baseline: 2.6159x; 1.2055x over previous
import functools

import jax
import jax.numpy as jnp
from jax import lax
from jax.experimental import pallas as pl
from jax.experimental.pallas import tpu as pltpu
from jax.experimental.pallas import tpu_sc as plsc

_D = 64
_NC = 2
_NS = 16
_NW = _NC * _NS
_NBUF = 4


def kernel(batch, table):
    b = batch.shape[0]
    per_w = b // _NW
    idx = batch.astype(jnp.int32).reshape(_NW, per_w)

    tt = table.T

    mesh = plsc.VectorSubcoreMesh(core_axis_name="c", subcore_axis_name="s")
    cp = pltpu.CompilerParams(
        needs_layout_passes=False, disable_bounds_checks=True
    )

    @functools.partial(
        pl.kernel,
        mesh=mesh,
        compiler_params=cp,
        out_type=jax.ShapeDtypeStruct((b, _D), table.dtype),
        scratch_types=[
            pltpu.VMEM((per_w,), jnp.int32),
            pltpu.VMEM((_NBUF, _D, 128), jnp.float32),
            pltpu.VMEM((per_w // 2, _D), jnp.float32),
            pltpu.SemaphoreType.DMA((_NBUF,)),
        ],
    )
    def gather_kernel(tt_hbm, idx_hbm, out_hbm, idx_v, tiles_v, out_v, sems):
        wid = lax.axis_index("s") * _NC + lax.axis_index("c")
        pltpu.sync_copy(idx_hbm.at[wid], idx_v)

        iota16 = lax.iota(jnp.int32, 16)
        half_w = per_w // 2

        def fetch(r, slot):
            c128 = pl.multiple_of((r >> 7) * 128, 128)
            pltpu.async_copy(
                tt_hbm.at[:, pl.ds(c128, 128)],
                tiles_v.at[slot],
                sems.at[slot],
            )

        def wait(slot):
            pltpu.make_async_copy(
                tt_hbm.at[:, pl.ds(0, 128)],
                tiles_v.at[slot],
                sems.at[slot],
            ).wait()

        for half in range(2):
            base = half * half_w
            jv0 = idx_v[pl.ds(base, 16)]
            for t in range(_NBUF):
                fetch(jv0[t], t)

            @pl.loop(0, half_w, step=16)
            def _(j0, base=base):
                jv = idx_v[pl.ds(base + j0, 16)]
                nxt = jnp.minimum(base + j0 + 16, per_w - 16)
                jn = idx_v[pl.ds(nxt, 16)]
                for i in range(16):
                    slot = i % _NBUF
                    wait(slot)
                    lane = jnp.broadcast_to(jv[i] & 127, (16,))
                    for q in range(4):
                        vals = plsc.load_gather(
                            tiles_v.at[slot], [iota16 + 16 * q, lane]
                        )
                        out_v[j0 + i, pl.ds(16 * q, 16)] = vals
                    nr = jv[i + _NBUF] if i < 16 - _NBUF else jn[i - (16 - _NBUF)]

                    @pl.when(j0 + i + _NBUF < half_w)
                    def _(nr=nr, slot=slot):
                        fetch(nr, slot)

            pltpu.sync_copy(
                out_v,
                out_hbm.at[pl.ds(wid * per_w + base, half_w)],
            )

    out = gather_kernel(tt, idx)
    return out.reshape(b, 1, _D)


# speedup vs baseline: 2.6967x; 1.0309x over previous
import functools

import jax
import jax.numpy as jnp
from jax import lax
from jax.experimental import pallas as pl
from jax.experimental.pallas import tpu as pltpu
from jax.experimental.pallas import tpu_sc as plsc

_D = 64
_NC = 2
_NS = 16
_NW = _NC * _NS
_NBUF = 4


def kernel(batch, table):
    b = batch.shape[0]
    per_w = b // _NW
    idx = batch.astype(jnp.int32).reshape(_NW, per_w)

    tt = table.T

    mesh = plsc.VectorSubcoreMesh(core_axis_name="c", subcore_axis_name="s")
    cp = pltpu.CompilerParams(
        needs_layout_passes=False, disable_bounds_checks=True
    )

    @functools.partial(
        pl.kernel,
        mesh=mesh,
        compiler_params=cp,
        out_type=jax.ShapeDtypeStruct((_D, b), table.dtype),
        scratch_types=[
            pltpu.VMEM((per_w,), jnp.int32),
            pltpu.VMEM((_NBUF, _D, 128), jnp.float32),
            pltpu.VMEM((_D, per_w // 2), jnp.float32),
            pltpu.SemaphoreType.DMA((_NBUF,)),
        ],
    )
    def gather_kernel(tt_hbm, idx_hbm, out_hbm, idx_v, tiles_v, out_v, sems):
        wid = lax.axis_index("s") * _NC + lax.axis_index("c")
        pltpu.sync_copy(idx_hbm.at[wid], idx_v)

        iota16 = lax.iota(jnp.int32, 16)
        half_w = per_w // 2

        def fetch(r, slot):
            c128 = pl.multiple_of((r >> 7) * 128, 128)
            pltpu.async_copy(
                tt_hbm.at[:, pl.ds(c128, 128)],
                tiles_v.at[slot],
                sems.at[slot],
            )

        def wait(slot):
            pltpu.make_async_copy(
                tt_hbm.at[:, pl.ds(0, 128)],
                tiles_v.at[slot],
                sems.at[slot],
            ).wait()

        for half in range(2):
            base = half * half_w
            jv0 = idx_v[pl.ds(base, 16)]
            for t in range(_NBUF):
                fetch(jv0[t], t)

            @pl.loop(0, half_w, step=16)
            def _(j0, base=base):
                jv = idx_v[pl.ds(base + j0, 16)]
                nxt = jnp.minimum(base + j0 + 16, per_w - 16)
                jn = idx_v[pl.ds(nxt, 16)]
                for i in range(16):
                    slot = i % _NBUF
                    wait(slot)
                    lane = jnp.broadcast_to(jv[i] & 127, (16,))
                    col = jnp.broadcast_to(j0 + i, (16,))
                    for q in range(4):
                        vals = plsc.load_gather(
                            tiles_v.at[slot], [iota16 + 16 * q, lane]
                        )
                        plsc.store_scatter(
                            out_v, [iota16 + 16 * q, col], vals
                        )
                    nr = jv[i + _NBUF] if i < 16 - _NBUF else jn[i - (16 - _NBUF)]

                    @pl.when(j0 + i + _NBUF < half_w)
                    def _(nr=nr, slot=slot):
                        fetch(nr, slot)

            pltpu.sync_copy(
                out_v,
                out_hbm.at[:, pl.ds(wid * per_w + base, half_w)],
            )

    out = gather_kernel(tt, idx)
    return out.T.reshape(b, 1, _D)
